# baseline (speedup 1.0000x reference)
import functools

import jax
import jax.numpy as jnp
from jax import lax
from jax.experimental import pallas as pl
from jax.experimental.pallas import tpu as pltpu
from jax.experimental.pallas import tpu_sc as plsc

BATCH = 16384
FPP = 32
FT_IN = 768
FT_OUT = 512
BUCKETS = 8
NNZ = BATCH * FPP
NSPLIT = 2
HALF = BATCH // NSPLIT

NC, NS = 2, 16
NW = NC * NS
ROWS_PER_W = HALF // NW
RCHUNK = 64
NCH = ROWS_PER_W // RCHUNK
IDX_CHUNK = RCHUNK * FPP
BOARD_WORDS = RCHUNK * FT_IN

BR = 512
GRID = HALF // BR


def _sc_boards_body(stm_idx_hbm, nstm_idx_hbm, vals_hbm,
                    bstm_hbm, bnstm_hbm,
                    idx_v0, idx_v1, val_v0, val_v1, board_v, bf_v0, bf_v1,
                    sem_in0, sem_in1, sem_out0, sem_out1):
    wid = lax.axis_index("s") * NC + lax.axis_index("c")
    zeros16 = jnp.zeros((16,), jnp.float32)
    idx_v = (idx_v0, idx_v1)
    val_v = (val_v0, val_v1)
    bf_v = (bf_v0, bf_v1)
    sem_in = (sem_in0, sem_in1)
    sem_out = (sem_out0, sem_out1)

    chunks = [(side, ch) for side in (0, 1) for ch in range(NCH)]

    def start_load(t):
        side, ch = chunks[t]
        b = t % 2
        idx_hbm = (stm_idx_hbm, nstm_idx_hbm)[side]
        slab = (wid * ROWS_PER_W + ch * RCHUNK) * FPP
        h_i = pltpu.make_async_copy(idx_hbm.at[pl.ds(slab, IDX_CHUNK)],
                                    idx_v[b], sem_in[b])
        h_v = pltpu.make_async_copy(vals_hbm.at[pl.ds(slab, IDX_CHUNK)],
                                    val_v[b], sem_in[b])
        h_i.start()
        h_v.start()
        return (h_i, h_v)

    def start_out(t):
        side, ch = chunks[t]
        b = t % 2
        out_hbm = (bstm_hbm, bnstm_hbm)[side]
        base = wid * ROWS_PER_W + ch * RCHUNK
        h = pltpu.make_async_copy(bf_v[b],
                                  out_hbm.at[pl.ds(base, RCHUNK)],
                                  sem_out[b])
        h.start()
        return h

    def zero_body(r, c):
        for u in range(FT_IN // 16):
            board_v[r, pl.ds(u * 16, 16)] = zeros16
        return c

    lax.fori_loop(0, RCHUNK, zero_body, 0)

    himask = jnp.full((16,), -0x10000, jnp.int32)

    loads = {0: start_load(0)}
    outs = {}
    for t in range(len(chunks)):
        b = t % 2
        if t + 1 < len(chunks):
            loads[t + 1] = start_load(t + 1)
        for h in loads.pop(t):
            h.wait()
        bv, iv, vv, bfv = board_v, idx_v[b], val_v[b], bf_v[b]

        def scat_body(i, c):
            for u in range(4):
                k = i * 4 + u
                row = jnp.full((16,), k // 2, jnp.int32)
                col = iv[pl.ds(k * 16, 16)]
                val = vv[pl.ds(k * 16, 16)]
                plsc.addupdate_scatter(bv, [row, col], val)
            return c

        lax.fori_loop(0, IDX_CHUNK // (4 * 16), scat_body, 0)

        if t - 2 in outs:
            outs.pop(t - 2).wait()

        def pack_body(r, c):
            for u in range(FT_IN // 32):
                a = plsc.bitcast(bv[r, pl.ds(u * 32, 16)], jnp.int32)
                b2 = plsc.bitcast(bv[r, pl.ds(u * 32 + 16, 16)], jnp.int32)
                comb = lax.shift_right_logical(a, 16) | (b2 & himask)
                bfv[r, pl.ds(u * 32, 32)] = plsc.bitcast(comb, jnp.bfloat16)
            return c

        lax.fori_loop(0, RCHUNK, pack_body, 0)

        outs[t] = start_out(t)

        def unscat_body(i, c):
            for u in range(4):
                k = i * 4 + u
                row = jnp.full((16,), k // 2, jnp.int32)
                col = iv[pl.ds(k * 16, 16)]
                plsc.store_scatter(bv, [row, col], zeros16)
            return c

        lax.fori_loop(0, IDX_CHUNK // (4 * 16), unscat_body, 0)
    for t in sorted(outs):
        outs.pop(t).wait()


_sc_boards = functools.partial(
    pl.kernel,
    out_type=[jax.ShapeDtypeStruct((HALF, FT_IN), jnp.bfloat16),
              jax.ShapeDtypeStruct((HALF, FT_IN), jnp.bfloat16)],
    mesh=plsc.VectorSubcoreMesh(core_axis_name="c", subcore_axis_name="s"),
    compiler_params=pltpu.CompilerParams(needs_layout_passes=False),
    scratch_types=[
        pltpu.VMEM((IDX_CHUNK,), jnp.int32),
        pltpu.VMEM((IDX_CHUNK,), jnp.int32),
        pltpu.VMEM((IDX_CHUNK,), jnp.float32),
        pltpu.VMEM((IDX_CHUNK,), jnp.float32),
        pltpu.VMEM((RCHUNK, FT_IN), jnp.float32),
        pltpu.VMEM((RCHUNK, FT_IN), jnp.bfloat16),
        pltpu.VMEM((RCHUNK, FT_IN), jnp.bfloat16),
        pltpu.SemaphoreType.DMA,
        pltpu.SemaphoreType.DMA,
        pltpu.SemaphoreType.DMA,
        pltpu.SemaphoreType.DMA,
    ],
)(_sc_boards_body)


def _tc_body(bstm_ref, bnstm_ref, buckets_ref, wft_ref, bft_ref,
             wout_ref, bout_ref, out_ref):
    wft = wft_ref[...]
    stm_ft = jnp.dot(bstm_ref[...], wft,
                     preferred_element_type=jnp.float32) + bft_ref[...]
    nstm_ft = jnp.dot(bnstm_ref[...], wft,
                      preferred_element_type=jnp.float32) + bft_ref[...]
    h0 = jnp.clip(stm_ft, 0.0, 1.0)
    h1 = jnp.clip(nstm_ft, 0.0, 1.0)
    l1 = (jnp.dot(h0, wout_ref[:FT_OUT], preferred_element_type=jnp.float32)
          + jnp.dot(h1, wout_ref[FT_OUT:], preferred_element_type=jnp.float32)
          + bout_ref[...])
    b = buckets_ref[0]
    onehot = (b == lax.broadcasted_iota(jnp.int32, (BR, BUCKETS), 1))
    val = jnp.sum(jnp.where(onehot, l1, 0.0), axis=1, keepdims=True)
    out_ref[...] = jax.nn.sigmoid(val)


def _tc_forward(bstm, bnstm, bkt, W_ft, b_ft, W_out, b_out):
    return pl.pallas_call(
        _tc_body,
        grid=(GRID,),
        in_specs=[
            pl.BlockSpec((BR, FT_IN), lambda i: (i, 0)),
            pl.BlockSpec((BR, FT_IN), lambda i: (i, 0)),
            pl.BlockSpec((1, BR, 1), lambda i: (i, 0, 0)),
            pl.BlockSpec((FT_IN, FT_OUT), lambda i: (0, 0)),
            pl.BlockSpec((1, FT_OUT), lambda i: (0, 0)),
            pl.BlockSpec((2 * FT_OUT, BUCKETS), lambda i: (0, 0)),
            pl.BlockSpec((1, BUCKETS), lambda i: (0, 0)),
        ],
        out_specs=pl.BlockSpec((BR, 1), lambda i: (i, 0)),
        out_shape=jax.ShapeDtypeStruct((HALF, 1), jnp.float32),
    )(bstm, bnstm, bkt, W_ft, b_ft, W_out, b_out)


def kernel(stm_indices, nstm_indices, values, size, buckets, W_ft, b_ft,
           W_out, b_out):
    stm_cols = stm_indices[:, 1].astype(jnp.int32)
    nstm_cols = nstm_indices[:, 1].astype(jnp.int32)
    vals = values.astype(jnp.float32)
    bkt = buckets.astype(jnp.int32).reshape(NSPLIT, GRID, BR, 1)
    b_ft2 = b_ft.reshape(1, FT_OUT)
    b_out2 = b_out.reshape(1, BUCKETS)
    p = jnp.arange(FT_IN, dtype=jnp.int32)
    u, j = p // 32, p % 32
    src = u * 32 + (j // 2) + (j % 2) * 16
    wft_bf = W_ft[src].astype(jnp.bfloat16)

    boards = []
    for k in range(NSPLIT):
        s = slice(k * HALF * FPP, (k + 1) * HALF * FPP)
        boards.append(_sc_boards(stm_cols[s], nstm_cols[s], vals[s]))
    outs = [_tc_forward(bstm, bnstm, bkt[k], wft_bf, b_ft2, W_out, b_out2)
            for k, (bstm, bnstm) in enumerate(boards)]
    return jnp.concatenate(outs, axis=0)


# speedup vs baseline: 1.7944x; 1.7944x over previous
import functools

import jax
import jax.numpy as jnp
from jax import lax
from jax.experimental import pallas as pl
from jax.experimental.pallas import tpu as pltpu
from jax.experimental.pallas import tpu_sc as plsc

BATCH = 16384
FPP = 32
FT_IN = 768
FT_OUT = 512
BUCKETS = 8
NNZ = BATCH * FPP
NSPLIT = 2
HALF = BATCH // NSPLIT

NC, NS = 2, 16
NW = NC * NS
ROWS_PER_W = HALF // NW
RCHUNK = 64
NCH = ROWS_PER_W // RCHUNK
IDX_CHUNK = RCHUNK * FPP
BOARD_WORDS = RCHUNK * FT_IN

BR = 512
GRID = HALF // BR


def _sc_boards_body(stm_idx_hbm, nstm_idx_hbm, vals_hbm,
                    bstm_hbm, bnstm_hbm,
                    idx_v0, idx_v1, val_v0, val_v1, board_v0, board_v1,
                    sem_in0, sem_in1, sem_out0, sem_out1):
    wid = lax.axis_index("s") * NC + lax.axis_index("c")
    zeros16 = jnp.zeros((16,), jnp.float32)
    idx_v = (idx_v0, idx_v1)
    val_v = (val_v0, val_v1)
    board_v = (board_v0, board_v1)
    sem_in = (sem_in0, sem_in1)
    sem_out = (sem_out0, sem_out1)

    chunks = [(side, ch) for side in (0, 1) for ch in range(NCH)]

    def start_load(t):
        side, ch = chunks[t]
        b = t % 2
        idx_hbm = (stm_idx_hbm, nstm_idx_hbm)[side]
        slab = (wid * ROWS_PER_W + ch * RCHUNK) * FPP
        h_i = pltpu.make_async_copy(idx_hbm.at[pl.ds(slab, IDX_CHUNK)],
                                    idx_v[b], sem_in[b])
        h_v = pltpu.make_async_copy(vals_hbm.at[pl.ds(slab, IDX_CHUNK)],
                                    val_v[b], sem_in[b])
        h_i.start()
        h_v.start()
        return (h_i, h_v)

    def start_out(t):
        side, ch = chunks[t]
        b = t % 2
        out_hbm = (bstm_hbm, bnstm_hbm)[side]
        base = wid * ROWS_PER_W + ch * RCHUNK
        h = pltpu.make_async_copy(board_v[b],
                                  out_hbm.at[pl.ds(base, RCHUNK)],
                                  sem_out[b])
        h.start()
        return h

    loads = {0: start_load(0)}
    outs = {}
    for t in range(len(chunks)):
        b = t % 2
        if t + 1 < len(chunks):
            loads[t + 1] = start_load(t + 1)
        for h in loads.pop(t):
            h.wait()
        if t - 2 in outs:
            outs.pop(t - 2).wait()
        bv, iv, vv = board_v[b], idx_v[b], val_v[b]

        def zero_body(r, c):
            for u in range(FT_IN // 16):
                bv[r, pl.ds(u * 16, 16)] = zeros16
            return c

        lax.fori_loop(0, RCHUNK, zero_body, 0)

        def scat_body(i, c):
            for u in range(4):
                k = i * 4 + u
                row = jnp.full((16,), k // 2, jnp.int32)
                col = iv[pl.ds(k * 16, 16)]
                val = vv[pl.ds(k * 16, 16)]
                plsc.addupdate_scatter(bv, [row, col], val)
            return c

        lax.fori_loop(0, IDX_CHUNK // (4 * 16), scat_body, 0)

        outs[t] = start_out(t)
    for t in sorted(outs):
        outs.pop(t).wait()


_sc_boards = functools.partial(
    pl.kernel,
    out_type=[jax.ShapeDtypeStruct((HALF, FT_IN), jnp.float32),
              jax.ShapeDtypeStruct((HALF, FT_IN), jnp.float32)],
    mesh=plsc.VectorSubcoreMesh(core_axis_name="c", subcore_axis_name="s"),
    compiler_params=pltpu.CompilerParams(needs_layout_passes=False),
    scratch_types=[
        pltpu.VMEM((IDX_CHUNK,), jnp.int32),
        pltpu.VMEM((IDX_CHUNK,), jnp.int32),
        pltpu.VMEM((IDX_CHUNK,), jnp.float32),
        pltpu.VMEM((IDX_CHUNK,), jnp.float32),
        pltpu.VMEM((RCHUNK, FT_IN), jnp.float32),
        pltpu.VMEM((RCHUNK, FT_IN), jnp.float32),
        pltpu.SemaphoreType.DMA,
        pltpu.SemaphoreType.DMA,
        pltpu.SemaphoreType.DMA,
        pltpu.SemaphoreType.DMA,
    ],
)(_sc_boards_body)


def _tc_body(bstm_ref, bnstm_ref, buckets_ref, wft_ref, bft_ref,
             wout_ref, bout_ref, out_ref):
    wft = wft_ref[...]
    stm_ft = jnp.dot(bstm_ref[...], wft,
                     preferred_element_type=jnp.float32) + bft_ref[...]
    nstm_ft = jnp.dot(bnstm_ref[...], wft,
                      preferred_element_type=jnp.float32) + bft_ref[...]
    h0 = jnp.clip(stm_ft, 0.0, 1.0)
    h1 = jnp.clip(nstm_ft, 0.0, 1.0)
    l1 = (jnp.dot(h0, wout_ref[:FT_OUT], preferred_element_type=jnp.float32)
          + jnp.dot(h1, wout_ref[FT_OUT:], preferred_element_type=jnp.float32)
          + bout_ref[...])
    b = buckets_ref[0]
    onehot = (b == lax.broadcasted_iota(jnp.int32, (BR, BUCKETS), 1))
    val = jnp.sum(jnp.where(onehot, l1, 0.0), axis=1, keepdims=True)
    out_ref[...] = jax.nn.sigmoid(val)


def _tc_forward(bstm, bnstm, bkt, W_ft, b_ft, W_out, b_out):
    return pl.pallas_call(
        _tc_body,
        grid=(GRID,),
        in_specs=[
            pl.BlockSpec((BR, FT_IN), lambda i: (i, 0)),
            pl.BlockSpec((BR, FT_IN), lambda i: (i, 0)),
            pl.BlockSpec((1, BR, 1), lambda i: (i, 0, 0)),
            pl.BlockSpec((FT_IN, FT_OUT), lambda i: (0, 0)),
            pl.BlockSpec((1, FT_OUT), lambda i: (0, 0)),
            pl.BlockSpec((2 * FT_OUT, BUCKETS), lambda i: (0, 0)),
            pl.BlockSpec((1, BUCKETS), lambda i: (0, 0)),
        ],
        out_specs=pl.BlockSpec((BR, 1), lambda i: (i, 0)),
        out_shape=jax.ShapeDtypeStruct((HALF, 1), jnp.float32),
    )(bstm, bnstm, bkt, W_ft, b_ft, W_out, b_out)


def kernel(stm_indices, nstm_indices, values, size, buckets, W_ft, b_ft,
           W_out, b_out):
    stm_cols = stm_indices[:, 1].astype(jnp.int32)
    nstm_cols = nstm_indices[:, 1].astype(jnp.int32)
    vals = values.astype(jnp.float32)
    bkt = buckets.astype(jnp.int32).reshape(NSPLIT, GRID, BR, 1)
    b_ft2 = b_ft.reshape(1, FT_OUT)
    b_out2 = b_out.reshape(1, BUCKETS)

    boards = []
    for k in range(NSPLIT):
        s = slice(k * HALF * FPP, (k + 1) * HALF * FPP)
        boards.append(_sc_boards(stm_cols[s], nstm_cols[s], vals[s]))
    outs = [_tc_forward(bstm, bnstm, bkt[k], W_ft, b_ft2, W_out, b_out2)
            for k, (bstm, bnstm) in enumerate(boards)]
    return jnp.concatenate(outs, axis=0)


# speedup vs baseline: 1.8853x; 1.0506x over previous
import functools

import jax
import jax.numpy as jnp
from jax import lax
from jax.experimental import pallas as pl
from jax.experimental.pallas import tpu as pltpu
from jax.experimental.pallas import tpu_sc as plsc

BATCH = 16384
FPP = 32
FT_IN = 768
FT_OUT = 512
BUCKETS = 8
NNZ = BATCH * FPP
NSPLIT = 2
HALF = BATCH // NSPLIT

NC, NS = 2, 16
NW = NC * NS
ROWS_PER_W = HALF // NW
RCHUNK = 64
NCH = ROWS_PER_W // RCHUNK
IDX_CHUNK = RCHUNK * FPP
BOARD_WORDS = RCHUNK * FT_IN

BR = 1024
GRID = HALF // BR


def _sc_boards_body(stm_idx_hbm, nstm_idx_hbm, vals_hbm,
                    bstm_hbm, bnstm_hbm,
                    idx_v0, idx_v1, val_v0, val_v1, board_v0, board_v1,
                    sem_in0, sem_in1, sem_out0, sem_out1):
    wid = lax.axis_index("s") * NC + lax.axis_index("c")
    zeros16 = jnp.zeros((16,), jnp.float32)
    idx_v = (idx_v0, idx_v1)
    val_v = (val_v0, val_v1)
    board_v = (board_v0, board_v1)
    sem_in = (sem_in0, sem_in1)
    sem_out = (sem_out0, sem_out1)

    chunks = [(side, ch) for side in (0, 1) for ch in range(NCH)]

    def start_load(t):
        side, ch = chunks[t]
        b = t % 2
        idx_hbm = (stm_idx_hbm, nstm_idx_hbm)[side]
        slab = (wid * ROWS_PER_W + ch * RCHUNK) * FPP
        h_i = pltpu.make_async_copy(idx_hbm.at[pl.ds(slab, IDX_CHUNK)],
                                    idx_v[b], sem_in[b])
        h_v = pltpu.make_async_copy(vals_hbm.at[pl.ds(slab, IDX_CHUNK)],
                                    val_v[b], sem_in[b])
        h_i.start()
        h_v.start()
        return (h_i, h_v)

    def start_out(t):
        side, ch = chunks[t]
        b = t % 2
        out_hbm = (bstm_hbm, bnstm_hbm)[side]
        base = wid * ROWS_PER_W + ch * RCHUNK
        h = pltpu.make_async_copy(board_v[b],
                                  out_hbm.at[pl.ds(base, RCHUNK)],
                                  sem_out[b])
        h.start()
        return h

    loads = {0: start_load(0)}
    outs = {}
    for t in range(len(chunks)):
        b = t % 2
        if t + 1 < len(chunks):
            loads[t + 1] = start_load(t + 1)
        for h in loads.pop(t):
            h.wait()
        if t - 2 in outs:
            outs.pop(t - 2).wait()
        bv, iv, vv = board_v[b], idx_v[b], val_v[b]

        def zero_body(r, c):
            for u in range(FT_IN // 16):
                bv[r, pl.ds(u * 16, 16)] = zeros16
            return c

        lax.fori_loop(0, RCHUNK, zero_body, 0)

        def scat_body(i, c):
            for u in range(4):
                k = i * 4 + u
                row = jnp.full((16,), k // 2, jnp.int32)
                col = iv[pl.ds(k * 16, 16)]
                val = vv[pl.ds(k * 16, 16)]
                plsc.addupdate_scatter(bv, [row, col], val)
            return c

        lax.fori_loop(0, IDX_CHUNK // (4 * 16), scat_body, 0)

        outs[t] = start_out(t)
    for t in sorted(outs):
        outs.pop(t).wait()


_sc_boards = functools.partial(
    pl.kernel,
    out_type=[jax.ShapeDtypeStruct((HALF, FT_IN), jnp.float32),
              jax.ShapeDtypeStruct((HALF, FT_IN), jnp.float32)],
    mesh=plsc.VectorSubcoreMesh(core_axis_name="c", subcore_axis_name="s"),
    compiler_params=pltpu.CompilerParams(needs_layout_passes=False),
    scratch_types=[
        pltpu.VMEM((IDX_CHUNK,), jnp.int32),
        pltpu.VMEM((IDX_CHUNK,), jnp.int32),
        pltpu.VMEM((IDX_CHUNK,), jnp.float32),
        pltpu.VMEM((IDX_CHUNK,), jnp.float32),
        pltpu.VMEM((RCHUNK, FT_IN), jnp.float32),
        pltpu.VMEM((RCHUNK, FT_IN), jnp.float32),
        pltpu.SemaphoreType.DMA,
        pltpu.SemaphoreType.DMA,
        pltpu.SemaphoreType.DMA,
        pltpu.SemaphoreType.DMA,
    ],
)(_sc_boards_body)


def _tc_body(bstm_ref, bnstm_ref, buckets_ref, wft_ref, bft_ref,
             wout_ref, bout_ref, out_ref):
    wft = wft_ref[...]
    stm_ft = jnp.dot(bstm_ref[...], wft,
                     preferred_element_type=jnp.float32) + bft_ref[...]
    nstm_ft = jnp.dot(bnstm_ref[...], wft,
                      preferred_element_type=jnp.float32) + bft_ref[...]
    h0 = jnp.clip(stm_ft, 0.0, 1.0)
    h1 = jnp.clip(nstm_ft, 0.0, 1.0)
    l1 = (jnp.dot(h0, wout_ref[:FT_OUT], preferred_element_type=jnp.float32)
          + jnp.dot(h1, wout_ref[FT_OUT:], preferred_element_type=jnp.float32)
          + bout_ref[...])
    b = buckets_ref[0]
    onehot = (b == lax.broadcasted_iota(jnp.int32, (BR, BUCKETS), 1))
    val = jnp.sum(jnp.where(onehot, l1, 0.0), axis=1, keepdims=True)
    out_ref[...] = jax.nn.sigmoid(val)


def _tc_forward(bstm, bnstm, bkt, W_ft, b_ft, W_out, b_out):
    return pl.pallas_call(
        _tc_body,
        grid=(GRID,),
        in_specs=[
            pl.BlockSpec((BR, FT_IN), lambda i: (i, 0)),
            pl.BlockSpec((BR, FT_IN), lambda i: (i, 0)),
            pl.BlockSpec((1, BR, 1), lambda i: (i, 0, 0)),
            pl.BlockSpec((FT_IN, FT_OUT), lambda i: (0, 0)),
            pl.BlockSpec((1, FT_OUT), lambda i: (0, 0)),
            pl.BlockSpec((2 * FT_OUT, BUCKETS), lambda i: (0, 0)),
            pl.BlockSpec((1, BUCKETS), lambda i: (0, 0)),
        ],
        out_specs=pl.BlockSpec((BR, 1), lambda i: (i, 0)),
        out_shape=jax.ShapeDtypeStruct((HALF, 1), jnp.float32),
    )(bstm, bnstm, bkt, W_ft, b_ft, W_out, b_out)


def kernel(stm_indices, nstm_indices, values, size, buckets, W_ft, b_ft,
           W_out, b_out):
    stm_cols = stm_indices[:, 1].astype(jnp.int32)
    nstm_cols = nstm_indices[:, 1].astype(jnp.int32)
    vals = values.astype(jnp.float32)
    bkt = buckets.astype(jnp.int32).reshape(NSPLIT, GRID, BR, 1)
    b_ft2 = b_ft.reshape(1, FT_OUT)
    b_out2 = b_out.reshape(1, BUCKETS)

    boards = []
    for k in range(NSPLIT):
        s = slice(k * HALF * FPP, (k + 1) * HALF * FPP)
        boards.append(_sc_boards(stm_cols[s], nstm_cols[s], vals[s]))
    outs = [_tc_forward(bstm, bnstm, bkt[k], W_ft, b_ft2, W_out, b_out2)
            for k, (bstm, bnstm) in enumerate(boards)]
    return jnp.concatenate(outs, axis=0)


# speedup vs baseline: 1.8927x; 1.0039x over previous
import functools

import jax
import jax.numpy as jnp
from jax import lax
from jax.experimental import pallas as pl
from jax.experimental.pallas import tpu as pltpu
from jax.experimental.pallas import tpu_sc as plsc

BATCH = 16384
FPP = 32
FT_IN = 768
FT_OUT = 512
BUCKETS = 8
NNZ = BATCH * FPP
NSPLIT = 2
HALF = BATCH // NSPLIT

NC, NS = 2, 16
NW = NC * NS
ROWS_PER_W = HALF // NW
RCHUNK = 64
NCH = ROWS_PER_W // RCHUNK
IDX_CHUNK = RCHUNK * FPP
BOARD_WORDS = RCHUNK * FT_IN

BR = 2048
GRID = HALF // BR


def _sc_boards_body(stm_idx_hbm, nstm_idx_hbm, vals_hbm,
                    bstm_hbm, bnstm_hbm,
                    idx_v0, idx_v1, val_v0, val_v1, board_v0, board_v1,
                    sem_in0, sem_in1, sem_out0, sem_out1):
    wid = lax.axis_index("s") * NC + lax.axis_index("c")
    zeros16 = jnp.zeros((16,), jnp.float32)
    idx_v = (idx_v0, idx_v1)
    val_v = (val_v0, val_v1)
    board_v = (board_v0, board_v1)
    sem_in = (sem_in0, sem_in1)
    sem_out = (sem_out0, sem_out1)

    chunks = [(side, ch) for side in (0, 1) for ch in range(NCH)]

    def start_load(t):
        side, ch = chunks[t]
        b = t % 2
        idx_hbm = (stm_idx_hbm, nstm_idx_hbm)[side]
        slab = (wid * ROWS_PER_W + ch * RCHUNK) * FPP
        h_i = pltpu.make_async_copy(idx_hbm.at[pl.ds(slab, IDX_CHUNK)],
                                    idx_v[b], sem_in[b])
        h_v = pltpu.make_async_copy(vals_hbm.at[pl.ds(slab, IDX_CHUNK)],
                                    val_v[b], sem_in[b])
        h_i.start()
        h_v.start()
        return (h_i, h_v)

    def start_out(t):
        side, ch = chunks[t]
        b = t % 2
        out_hbm = (bstm_hbm, bnstm_hbm)[side]
        base = wid * ROWS_PER_W + ch * RCHUNK
        h = pltpu.make_async_copy(board_v[b],
                                  out_hbm.at[pl.ds(base, RCHUNK)],
                                  sem_out[b])
        h.start()
        return h

    loads = {0: start_load(0)}
    outs = {}
    for t in range(len(chunks)):
        b = t % 2
        if t + 1 < len(chunks):
            loads[t + 1] = start_load(t + 1)
        for h in loads.pop(t):
            h.wait()
        if t - 2 in outs:
            outs.pop(t - 2).wait()
        bv, iv, vv = board_v[b], idx_v[b], val_v[b]

        def zero_body(r, c):
            for u in range(FT_IN // 16):
                bv[r, pl.ds(u * 16, 16)] = zeros16
            return c

        lax.fori_loop(0, RCHUNK, zero_body, 0)

        def scat_body(i, c):
            for u in range(4):
                k = i * 4 + u
                row = jnp.full((16,), k // 2, jnp.int32)
                col = iv[pl.ds(k * 16, 16)]
                val = vv[pl.ds(k * 16, 16)]
                plsc.addupdate_scatter(bv, [row, col], val)
            return c

        lax.fori_loop(0, IDX_CHUNK // (4 * 16), scat_body, 0)

        outs[t] = start_out(t)
    for t in sorted(outs):
        outs.pop(t).wait()


_sc_boards = functools.partial(
    pl.kernel,
    out_type=[jax.ShapeDtypeStruct((HALF, FT_IN), jnp.float32),
              jax.ShapeDtypeStruct((HALF, FT_IN), jnp.float32)],
    mesh=plsc.VectorSubcoreMesh(core_axis_name="c", subcore_axis_name="s"),
    compiler_params=pltpu.CompilerParams(needs_layout_passes=False),
    scratch_types=[
        pltpu.VMEM((IDX_CHUNK,), jnp.int32),
        pltpu.VMEM((IDX_CHUNK,), jnp.int32),
        pltpu.VMEM((IDX_CHUNK,), jnp.float32),
        pltpu.VMEM((IDX_CHUNK,), jnp.float32),
        pltpu.VMEM((RCHUNK, FT_IN), jnp.float32),
        pltpu.VMEM((RCHUNK, FT_IN), jnp.float32),
        pltpu.SemaphoreType.DMA,
        pltpu.SemaphoreType.DMA,
        pltpu.SemaphoreType.DMA,
        pltpu.SemaphoreType.DMA,
    ],
)(_sc_boards_body)


def _tc_body(bstm_ref, bnstm_ref, buckets_ref, wft_ref, bft_ref,
             wout_ref, bout_ref, out_ref):
    wft = wft_ref[...]
    stm_ft = jnp.dot(bstm_ref[...], wft,
                     preferred_element_type=jnp.float32) + bft_ref[...]
    nstm_ft = jnp.dot(bnstm_ref[...], wft,
                      preferred_element_type=jnp.float32) + bft_ref[...]
    h0 = jnp.clip(stm_ft, 0.0, 1.0)
    h1 = jnp.clip(nstm_ft, 0.0, 1.0)
    l1 = (jnp.dot(h0, wout_ref[:FT_OUT], preferred_element_type=jnp.float32)
          + jnp.dot(h1, wout_ref[FT_OUT:], preferred_element_type=jnp.float32)
          + bout_ref[...])
    b = buckets_ref[0]
    onehot = (b == lax.broadcasted_iota(jnp.int32, (BR, BUCKETS), 1))
    val = jnp.sum(jnp.where(onehot, l1, 0.0), axis=1, keepdims=True)
    out_ref[...] = jax.nn.sigmoid(val)


def _tc_forward(bstm, bnstm, bkt, W_ft, b_ft, W_out, b_out):
    return pl.pallas_call(
        _tc_body,
        grid=(GRID,),
        in_specs=[
            pl.BlockSpec((BR, FT_IN), lambda i: (i, 0)),
            pl.BlockSpec((BR, FT_IN), lambda i: (i, 0)),
            pl.BlockSpec((1, BR, 1), lambda i: (i, 0, 0)),
            pl.BlockSpec((FT_IN, FT_OUT), lambda i: (0, 0)),
            pl.BlockSpec((1, FT_OUT), lambda i: (0, 0)),
            pl.BlockSpec((2 * FT_OUT, BUCKETS), lambda i: (0, 0)),
            pl.BlockSpec((1, BUCKETS), lambda i: (0, 0)),
        ],
        out_specs=pl.BlockSpec((BR, 1), lambda i: (i, 0)),
        out_shape=jax.ShapeDtypeStruct((HALF, 1), jnp.float32),
    )(bstm, bnstm, bkt, W_ft, b_ft, W_out, b_out)


def kernel(stm_indices, nstm_indices, values, size, buckets, W_ft, b_ft,
           W_out, b_out):
    stm_cols = stm_indices[:, 1].astype(jnp.int32)
    nstm_cols = nstm_indices[:, 1].astype(jnp.int32)
    vals = values.astype(jnp.float32)
    bkt = buckets.astype(jnp.int32).reshape(NSPLIT, GRID, BR, 1)
    b_ft2 = b_ft.reshape(1, FT_OUT)
    b_out2 = b_out.reshape(1, BUCKETS)

    boards = []
    for k in range(NSPLIT):
        s = slice(k * HALF * FPP, (k + 1) * HALF * FPP)
        boards.append(_sc_boards(stm_cols[s], nstm_cols[s], vals[s]))
    outs = [_tc_forward(bstm, bnstm, bkt[k], W_ft, b_ft2, W_out, b_out2)
            for k, (bstm, bnstm) in enumerate(boards)]
    return jnp.concatenate(outs, axis=0)
